# baseline (speedup 1.0000x reference)
import jax
import jax.numpy as jnp
from jax import lax
from jax.experimental import pallas as pl
from jax.experimental.pallas import tpu as pltpu
from jax.experimental.pallas import tpu_sc as plsc

NC = 2
NS = 16
NW = NC * NS
D = 64
CH = 128
L = 16


NB = 7813


CB = 256
NCB = 3907


def _conv_body(wT_hbm, w2_hbm, bvA, bvB, tvA, tvB, gA, gB, sA, sB):
    wid = lax.axis_index("s") * NC + lax.axis_index("c")
    ivec = lax.iota(jnp.int32, L)

    def fire_read(c, bv, sem):
        for t in range(8):
            pltpu.async_copy(
                wT_hbm.at[pl.ds(8 * t, 8), pl.ds(c * CB, CB)],
                bv.at[pl.ds(8 * t, 8), pl.ds(0, CB)], sem)

    def wait_read(bv, sem):
        for t in range(8):
            pltpu.make_async_copy(
                wT_hbm.at[pl.ds(8 * t, 8), pl.ds(0, CB)],
                bv.at[pl.ds(8 * t, 8), pl.ds(0, CB)], sem).wait()

    dvecs = [lax.bitwise_and(ivec + cg * L, D - 1) for cg in range(2 * D // L)]

    def transpose(bv, tv):
        @pl.loop(0, CB // 2, unroll=4)
        def _(r):
            for cg in range(2 * D // L):
                i = 2 * r + (1 if cg >= D // L else 0)
                vals = plsc.load_gather(bv, [dvecs[cg], lax.broadcast(i, (L,))])
                tv[r, pl.ds(cg * L, L)] = vals

    def fire_store(c, tv, sem):
        pltpu.async_copy(
            tv.at[:, pl.ds(0, 2 * D)], w2_hbm.at[pl.ds(c * (CB // 2), CB // 2), :], sem)

    def wait_store(tv, sem):
        pltpu.make_async_copy(
            tv.at[:, pl.ds(0, 2 * D)],
            w2_hbm.at[pl.ds(0, CB // 2), :], sem).wait()

    def blk(g):
        return wid + NW * g

    fire_read(blk(0), bvA, gA)
    n_full = NCB - 1

    @pl.loop(0, (n_full + NW - 1) // NW // 2 + 1)
    def _(i):
        ca = blk(2 * i)
        cb = blk(2 * i + 1)

        @pl.when(ca < n_full)
        def _():
            wait_read(bvA, gA)

        @pl.when(cb < n_full)
        def _():
            fire_read(cb, bvB, gB)

        @pl.when(jnp.logical_and(ca < n_full, i > 0))
        def _():
            wait_store(tvA, sA)

        @pl.when(ca < n_full)
        def _():
            transpose(bvA, tvA)
            fire_store(ca, tvA, sA)

        @pl.when(cb < n_full)
        def _():
            wait_read(bvB, gB)

        @pl.when(blk(2 * i + 2) < n_full)
        def _():
            fire_read(blk(2 * i + 2), bvA, gA)

        @pl.when(jnp.logical_and(cb < n_full, i > 0))
        def _():
            wait_store(tvB, sB)

        @pl.when(cb < n_full)
        def _():
            transpose(bvB, tvB)
            fire_store(cb, tvB, sB)

    wait_store(tvA, sA)
    wait_store(tvB, sB)

    @pl.when(wid == 0)
    def _():
        tail0 = (NCB - 1) * jnp.int32(CB)
        for t in range(8):
            pltpu.sync_copy(
                wT_hbm.at[pl.ds(8 * t, 8), pl.ds(tail0, 128)],
                bvA.at[pl.ds(8 * t, 8), pl.ds(0, 128)])
        @pl.loop(0, 32, unroll=4)
        def _(r):
            for cg in range(2 * D // L):
                i = 2 * r + (1 if cg >= D // L else 0)
                vals = plsc.load_gather(
                    bvA, [dvecs[cg], lax.broadcast(i, (L,))])
                tvA[r, pl.ds(cg * L, L)] = vals
        r0 = (NCB - 1) * (CB // 2)
        pltpu.sync_copy(
            tvA.at[pl.ds(0, 32), pl.ds(0, 2 * D)],
            w2_hbm.at[pl.ds(r0, 32), :])


def _body(x_hbm, w_hbm, out_hbm, jv, ov, rawA, rawB, tbA, tbB, gA, gB, sA, sB):
    n_b1 = x_hbm.shape[0]
    wid = lax.axis_index("s") * NC + lax.axis_index("c")
    b0 = wid * CH

    pltpu.sync_copy(x_hbm.at[:, pl.ds(b0, CH)], jv)

    @pl.loop(0, n_b1)
    def _(b1):
        for g in range(CH // L):
            sl = pl.ds(g * L, L)
            v = jv[b1, sl]
            ov[b1, sl] = lax.shift_left(lax.bitwise_and(v, 1), 6)
            jv[b1, sl] = lax.shift_right_logical(v, 1)

    def fire_gather(b1, raw, sem):
        pltpu.async_copy(w_hbm.at[jv.at[b1]], raw, sem)

    def wait_gather(raw, sem):
        pltpu.make_async_copy(w_hbm.at[jv.at[0]], raw, sem).wait()

    tvec0 = lax.iota(jnp.int32, L)

    def transpose(b1, raw, tb):
        @pl.loop(0, CH // L)
        def _(g):
            tvec = tvec0 + g * L
            offv = ov[b1, pl.ds(g * L, L)]

            @pl.loop(0, D, unroll=16)
            def _(d):
                evec = lax.bitwise_and(tvec + d, D - 1)
                vals = plsc.load_gather(raw, [tvec, offv + evec])
                plsc.store_scatter(tb, [evec, tvec], vals)

    def fire_store(b1, tb, sem):
        pltpu.async_copy(
            tb.at[:, pl.ds(0, CH)], out_hbm.at[b1, :, pl.ds(b0, CH)], sem)

    def wait_store(tb, sem):
        pltpu.make_async_copy(
            tb.at[:, pl.ds(0, CH)], out_hbm.at[0, :, pl.ds(b0, CH)], sem
        ).wait()

    fire_gather(0, rawA, gA)

    @pl.loop(0, n_b1 // 2)
    def _(i):
        b1a = 2 * i
        b1b = 2 * i + 1
        wait_gather(rawA, gA)
        fire_gather(b1b, rawB, gB)

        @pl.when(i > 0)
        def _():
            wait_store(tbA, sA)

        transpose(b1a, rawA, tbA)
        fire_store(b1a, tbA, sA)

        wait_gather(rawB, gB)

        @pl.when(b1b + 1 < n_b1)
        def _():
            fire_gather(b1b + 1, rawA, gA)

        @pl.when(i > 0)
        def _():
            wait_store(tbB, sB)

        transpose(b1b, rawB, tbB)
        fire_store(b1b, tbB, sB)

    wait_store(tbA, sA)
    wait_store(tbB, sB)


def kernel(x, weight):
    b0n, b1n = x.shape
    nvoc, d = weight.shape
    assert d == D and b0n == NW * CH
    xT = x.T.astype(jnp.int32)
    wT = weight.T

    mesh = plsc.VectorSubcoreMesh(core_axis_name="c", subcore_axis_name="s")
    w2 = pl.kernel(
        _conv_body,
        out_type=jax.ShapeDtypeStruct((nvoc // 2, 2 * D), jnp.float32),
        mesh=mesh,
        scratch_types=[
            pltpu.VMEM((D, CB + 1), jnp.float32),
            pltpu.VMEM((D, CB + 1), jnp.float32),
            pltpu.VMEM((CB // 2, 2 * D + 1), jnp.float32),
            pltpu.VMEM((CB // 2, 2 * D + 1), jnp.float32),
            pltpu.SemaphoreType.DMA,
            pltpu.SemaphoreType.DMA,
            pltpu.SemaphoreType.DMA,
            pltpu.SemaphoreType.DMA,
        ],
        compiler_params=pltpu.CompilerParams(
            use_tc_tiling_on_sc=True, needs_layout_passes=False),
    )(wT)
    outT = pl.kernel(
        _body,
        out_type=jax.ShapeDtypeStruct((b1n, D, b0n), jnp.float32),
        mesh=mesh,
        scratch_types=[
            pltpu.VMEM((b1n, CH), jnp.int32),
            pltpu.VMEM((b1n, CH), jnp.int32),
            pltpu.VMEM((CH, 2 * D), jnp.float32),
            pltpu.VMEM((CH, 2 * D), jnp.float32),
            pltpu.VMEM((D, CH + 2), jnp.float32),
            pltpu.VMEM((D, CH + 2), jnp.float32),
            pltpu.SemaphoreType.DMA,
            pltpu.SemaphoreType.DMA,
            pltpu.SemaphoreType.DMA,
            pltpu.SemaphoreType.DMA,
        ],
        compiler_params=pltpu.CompilerParams(
            use_tc_tiling_on_sc=True, needs_layout_passes=False),
    )(xT, w2)
    return jnp.transpose(outT, (2, 0, 1))


# speedup vs baseline: 1.9414x; 1.9414x over previous
import jax
import jax.numpy as jnp
from jax import lax
from jax.experimental import pallas as pl
from jax.experimental.pallas import tpu as pltpu
from jax.experimental.pallas import tpu_sc as plsc

NC = 2
NS = 16
NW = NC * NS
D = 64
CH = 128
L = 16


def _body(x_hbm, w_hbm, out_hbm, jv, ov, rawA, rawB, tbA, tbB, gA, gB, sA, sB):
    n_b1 = x_hbm.shape[0]
    wid = lax.axis_index("s") * NC + lax.axis_index("c")
    b0 = wid * CH

    pltpu.sync_copy(x_hbm.at[:, pl.ds(b0, CH)], jv)

    @pl.loop(0, n_b1)
    def _(b1):
        for g in range(CH // L):
            sl = pl.ds(g * L, L)
            v = jv[b1, sl]
            ov[b1, sl] = lax.shift_left(lax.bitwise_and(v, 1), 6)
            jv[b1, sl] = lax.shift_right_logical(v, 1)

    def fire_gather(b1, raw, sem):
        pltpu.async_copy(w_hbm.at[jv.at[b1]], raw, sem)

    def wait_gather(raw, sem):
        pltpu.make_async_copy(w_hbm.at[jv.at[0]], raw, sem).wait()

    tvec0 = lax.iota(jnp.int32, L)

    def transpose(b1, raw, tb):
        @pl.loop(0, CH // L)
        def _(g):
            tvec = tvec0 + g * L
            offv = ov[b1, pl.ds(g * L, L)]

            @pl.loop(0, D, unroll=32)
            def _(d):
                evec = lax.bitwise_and(tvec + d, D - 1)
                vals = plsc.load_gather(raw, [tvec, offv + evec])
                plsc.store_scatter(tb, [evec, tvec], vals)

    def fire_store(b1, tb, sem):
        pltpu.async_copy(
            tb.at[:, pl.ds(0, CH)], out_hbm.at[b1, :, pl.ds(b0, CH)], sem)

    def wait_store(tb, sem):
        pltpu.make_async_copy(
            tb.at[:, pl.ds(0, CH)], out_hbm.at[0, :, pl.ds(b0, CH)], sem
        ).wait()

    fire_gather(0, rawA, gA)

    @pl.loop(0, n_b1 // 2)
    def _(i):
        b1a = 2 * i
        b1b = 2 * i + 1
        wait_gather(rawA, gA)
        fire_gather(b1b, rawB, gB)

        @pl.when(i > 0)
        def _():
            wait_store(tbA, sA)

        transpose(b1a, rawA, tbA)
        fire_store(b1a, tbA, sA)

        wait_gather(rawB, gB)

        @pl.when(b1b + 1 < n_b1)
        def _():
            fire_gather(b1b + 1, rawA, gA)

        @pl.when(i > 0)
        def _():
            wait_store(tbB, sB)

        transpose(b1b, rawB, tbB)
        fire_store(b1b, tbB, sB)

    wait_store(tbA, sA)
    wait_store(tbB, sB)


def kernel(x, weight):
    b0n, b1n = x.shape
    nvoc, d = weight.shape
    assert d == D and b0n == NW * CH
    xT = x.T.astype(jnp.int32)
    w2 = weight.reshape(nvoc // 2, 2 * D)

    mesh = plsc.VectorSubcoreMesh(core_axis_name="c", subcore_axis_name="s")
    outT = pl.kernel(
        _body,
        out_type=jax.ShapeDtypeStruct((b1n, D, b0n), jnp.float32),
        mesh=mesh,
        scratch_types=[
            pltpu.VMEM((b1n, CH), jnp.int32),
            pltpu.VMEM((b1n, CH), jnp.int32),
            pltpu.VMEM((CH, 2 * D), jnp.float32),
            pltpu.VMEM((CH, 2 * D), jnp.float32),
            pltpu.VMEM((D, CH + 2), jnp.float32),
            pltpu.VMEM((D, CH + 2), jnp.float32),
            pltpu.SemaphoreType.DMA,
            pltpu.SemaphoreType.DMA,
            pltpu.SemaphoreType.DMA,
            pltpu.SemaphoreType.DMA,
        ],
        compiler_params=pltpu.CompilerParams(
            use_tc_tiling_on_sc=True, needs_layout_passes=False),
    )(xT, w2)
    return jnp.transpose(outT, (2, 0, 1))


# speedup vs baseline: 1.9508x; 1.0049x over previous
import jax
import jax.numpy as jnp
from jax import lax
from jax.experimental import pallas as pl
from jax.experimental.pallas import tpu as pltpu
from jax.experimental.pallas import tpu_sc as plsc

NC = 2
NS = 16
NW = NC * NS
D = 64
CH = 128
L = 16


def _body(x_hbm, w_hbm, out_hbm, jv, ov, rawA, rawB, tbA, tbB, gA, gB, sA, sB):
    n_b1 = x_hbm.shape[0]
    wid = lax.axis_index("s") * NC + lax.axis_index("c")
    b0 = wid * CH

    pltpu.sync_copy(x_hbm.at[:, pl.ds(b0, CH)], jv)

    @pl.loop(0, n_b1)
    def _(b1):
        for g in range(CH // L):
            sl = pl.ds(g * L, L)
            v = jv[b1, sl]
            ov[b1, sl] = lax.shift_left(lax.bitwise_and(v, 1), 6)
            jv[b1, sl] = lax.shift_right_logical(v, 1)

    def fire_gather(b1, raw, sem):
        pltpu.async_copy(w_hbm.at[jv.at[b1]], raw, sem)

    def wait_gather(raw, sem):
        pltpu.make_async_copy(w_hbm.at[jv.at[0]], raw, sem).wait()

    tvec0 = lax.iota(jnp.int32, L)

    def transpose(b1, raw, tb):
        @pl.loop(0, CH // L, unroll=2)
        def _(g):
            tvec = tvec0 + g * L
            offv = ov[b1, pl.ds(g * L, L)]

            @pl.loop(0, D, unroll=32)
            def _(d):
                evec = lax.bitwise_and(tvec + d, D - 1)
                vals = plsc.load_gather(raw, [tvec, offv + evec])
                plsc.store_scatter(tb, [evec, tvec], vals)

    def fire_store(b1, tb, sem):
        pltpu.async_copy(
            tb.at[:, pl.ds(0, CH)], out_hbm.at[b1, :, pl.ds(b0, CH)], sem)

    def wait_store(tb, sem):
        pltpu.make_async_copy(
            tb.at[:, pl.ds(0, CH)], out_hbm.at[0, :, pl.ds(b0, CH)], sem
        ).wait()

    fire_gather(0, rawA, gA)

    @pl.loop(0, n_b1 // 2)
    def _(i):
        b1a = 2 * i
        b1b = 2 * i + 1
        wait_gather(rawA, gA)
        fire_gather(b1b, rawB, gB)

        @pl.when(i > 0)
        def _():
            wait_store(tbA, sA)

        transpose(b1a, rawA, tbA)
        fire_store(b1a, tbA, sA)

        wait_gather(rawB, gB)

        @pl.when(b1b + 1 < n_b1)
        def _():
            fire_gather(b1b + 1, rawA, gA)

        @pl.when(i > 0)
        def _():
            wait_store(tbB, sB)

        transpose(b1b, rawB, tbB)
        fire_store(b1b, tbB, sB)

    wait_store(tbA, sA)
    wait_store(tbB, sB)


def kernel(x, weight):
    b0n, b1n = x.shape
    nvoc, d = weight.shape
    assert d == D and b0n == NW * CH
    xT = x.T.astype(jnp.int32)
    w2 = weight.reshape(nvoc // 2, 2 * D)

    mesh = plsc.VectorSubcoreMesh(core_axis_name="c", subcore_axis_name="s")
    outT = pl.kernel(
        _body,
        out_type=jax.ShapeDtypeStruct((b1n, D, b0n), jnp.float32),
        mesh=mesh,
        scratch_types=[
            pltpu.VMEM((b1n, CH), jnp.int32),
            pltpu.VMEM((b1n, CH), jnp.int32),
            pltpu.VMEM((CH, 2 * D), jnp.float32),
            pltpu.VMEM((CH, 2 * D), jnp.float32),
            pltpu.VMEM((D, CH + 2), jnp.float32),
            pltpu.VMEM((D, CH + 2), jnp.float32),
            pltpu.SemaphoreType.DMA,
            pltpu.SemaphoreType.DMA,
            pltpu.SemaphoreType.DMA,
            pltpu.SemaphoreType.DMA,
        ],
        compiler_params=pltpu.CompilerParams(
            use_tc_tiling_on_sc=True, needs_layout_passes=False),
    )(xT, w2)
    return jnp.transpose(outT, (2, 0, 1))


# speedup vs baseline: 1.9509x; 1.0001x over previous
import jax
import jax.numpy as jnp
from jax import lax
from jax.experimental import pallas as pl
from jax.experimental.pallas import tpu as pltpu
from jax.experimental.pallas import tpu_sc as plsc

NC = 2
NS = 16
NW = NC * NS
D = 64
CH = 128
L = 16


def _body(x_hbm, w_hbm, out_hbm, jv, ov, rawA, rawB, tbA, tbB, gA, gB, sA, sB):
    n_b1 = x_hbm.shape[0]
    wid = lax.axis_index("s") * NC + lax.axis_index("c")
    b0 = wid * CH

    pltpu.sync_copy(x_hbm.at[:, pl.ds(b0, CH)], jv)

    @pl.loop(0, n_b1)
    def _(b1):
        for g in range(CH // L):
            sl = pl.ds(g * L, L)
            v = jv[b1, sl]
            ov[b1, sl] = lax.shift_left(lax.bitwise_and(v, 1), 6)
            jv[b1, sl] = lax.shift_right_logical(v, 1)

    def fire_gather(b1, raw, sem):
        pltpu.async_copy(w_hbm.at[jv.at[b1]], raw, sem)

    def wait_gather(raw, sem):
        pltpu.make_async_copy(w_hbm.at[jv.at[0]], raw, sem).wait()

    tvec0 = lax.iota(jnp.int32, L)

    def transpose(b1, raw, tb):
        @pl.loop(0, CH // L, unroll=2)
        def _(g):
            tvec = tvec0 + g * L
            offv = ov[b1, pl.ds(g * L, L)]

            @pl.loop(0, D, unroll=32)
            def _(d):
                evec = lax.bitwise_and(tvec + d, D - 1)
                vals = plsc.load_gather(raw, [tvec, offv + evec])
                plsc.store_scatter(tb, [evec, tvec], vals)

    def fire_store(b1, tb, sem):
        pltpu.async_copy(
            tb.at[:, pl.ds(0, CH)], out_hbm.at[b1, :, pl.ds(b0, CH)], sem)

    def wait_store(tb, sem):
        pltpu.make_async_copy(
            tb.at[:, pl.ds(0, CH)], out_hbm.at[0, :, pl.ds(b0, CH)], sem
        ).wait()

    fire_gather(0, rawA, gA)

    @pl.loop(0, n_b1 // 2)
    def _(i):
        b1a = 2 * i
        b1b = 2 * i + 1
        fire_gather(b1b, rawB, gB)
        wait_gather(rawA, gA)

        @pl.when(i > 0)
        def _():
            wait_store(tbA, sA)

        transpose(b1a, rawA, tbA)
        fire_store(b1a, tbA, sA)

        @pl.when(b1b + 1 < n_b1)
        def _():
            fire_gather(b1b + 1, rawA, gA)

        wait_gather(rawB, gB)

        @pl.when(i > 0)
        def _():
            wait_store(tbB, sB)

        transpose(b1b, rawB, tbB)
        fire_store(b1b, tbB, sB)

    wait_store(tbA, sA)
    wait_store(tbB, sB)


def kernel(x, weight):
    b0n, b1n = x.shape
    nvoc, d = weight.shape
    assert d == D and b0n == NW * CH
    xT = x.T.astype(jnp.int32)
    w2 = weight.reshape(nvoc // 2, 2 * D)

    mesh = plsc.VectorSubcoreMesh(core_axis_name="c", subcore_axis_name="s")
    outT = pl.kernel(
        _body,
        out_type=jax.ShapeDtypeStruct((b1n, D, b0n), jnp.float32),
        mesh=mesh,
        scratch_types=[
            pltpu.VMEM((b1n, CH), jnp.int32),
            pltpu.VMEM((b1n, CH), jnp.int32),
            pltpu.VMEM((CH, 2 * D), jnp.float32),
            pltpu.VMEM((CH, 2 * D), jnp.float32),
            pltpu.VMEM((D, CH + 2), jnp.float32),
            pltpu.VMEM((D, CH + 2), jnp.float32),
            pltpu.SemaphoreType.DMA,
            pltpu.SemaphoreType.DMA,
            pltpu.SemaphoreType.DMA,
            pltpu.SemaphoreType.DMA,
        ],
        compiler_params=pltpu.CompilerParams(
            use_tc_tiling_on_sc=True, needs_layout_passes=False),
    )(xT, w2)
    return jnp.transpose(outT, (2, 0, 1))
